# baseline (speedup 1.0000x reference)
import math

import jax
import jax.numpy as jnp
from jax.experimental import pallas as pl
from jax.experimental.pallas import tpu as pltpu


def _conv_stage_kernel(x_ref, wb_ref, b1_ref, w2_ref, b2_ref, o_ref):
    GB = x_ref.shape[1]
    x = x_ref[...].astype(jnp.bfloat16)
    x5 = jnp.concatenate([x[kh:kh + 28] for kh in range(5)], axis=-1)
    y1 = jnp.dot(x5.reshape(28 * GB, 160), wb_ref[...],
                 preferred_element_type=jnp.float32)
    y1 = jnp.maximum(y1 + b1_ref[...], 0.0)
    y1 = y1.astype(jnp.bfloat16).reshape(28, GB, 896)

    tg = []
    for g in range(6):
        pg = jnp.concatenate(
            [y1[kh:kh + 24, :, 128 * g:128 * g + 256] for kh in range(5)],
            axis=-1)
        cg = jnp.dot(pg.reshape(24 * GB, 1280), w2_ref[...],
                     preferred_element_type=jnp.float32)
        cg = jnp.maximum(cg + b2_ref[...], 0.0)
        cg = cg.astype(jnp.bfloat16).reshape(24, GB, 256)
        t = cg[0:20]
        for d in range(1, 5):
            t = jnp.maximum(t, cg[d:d + 20])
        tg.append(t)

    for gp in range(5):
        s = jnp.concatenate([tg[gp], tg[gp + 1]], axis=-1)
        chunks = []
        for j in range(4):
            m = s[:, :, 64 * j:64 * j + 64]
            for d in range(1, 5):
                m = jnp.maximum(m, s[:, :, 64 * (j + d):64 * (j + d) + 64])
            chunks.append(m)
        o_ref[:, gp] = jnp.concatenate(chunks, axis=-1)


def _mlp_kernel(f_ref, w1_ref, b1_ref, w2_ref, b2_ref, o_ref, acc_ref):
    k = pl.program_id(0)
    Q = f_ref.shape[0]

    @pl.when(k == 0)
    def _():
        acc_ref[...] = jnp.zeros_like(acc_ref)

    part = jnp.dot(f_ref[0], w1_ref[k * Q],
                   preferred_element_type=jnp.float32)
    for qq in range(1, Q):
        part = part + jnp.dot(f_ref[qq], w1_ref[k * Q + qq],
                              preferred_element_type=jnp.float32)
    acc_ref[...] += part

    @pl.when(k == pl.num_programs(0) - 1)
    def _():
        h = jnp.maximum(acc_ref[...] + b1_ref[...], 0.0)
        logits = jnp.dot(h, w2_ref[...],
                         preferred_element_type=jnp.float32) + b2_ref[...]
        m = jnp.max(logits, axis=1, keepdims=True)
        e = jnp.exp(logits - m)
        o_ref[...] = e / jnp.sum(e, axis=1, keepdims=True)


def kernel(x, Wb, b1t, w2m, b2r, wl1m, bl1r, wl2m, bl2r):
    N = x.shape[0]
    xb = x.reshape(N, 32, 32).transpose(1, 0, 2)
    wbcat = Wb.reshape(160, 896).astype(jnp.bfloat16)

    w25 = w2m.reshape(5, 5, 32, 64)
    segs = [jnp.pad(w25, ((0, 0), (j, 3 - j), (0, 0), (0, 0)))
            for j in range(4)]
    W2p = jnp.stack(segs, axis=3).reshape(1280, 256).astype(jnp.bfloat16)
    b2p = jnp.tile(b2r, (1, 4))

    GB = math.gcd(N, 16)
    f = pl.pallas_call(
        _conv_stage_kernel,
        out_shape=jax.ShapeDtypeStruct((20, 5, N, 256), jnp.bfloat16),
        grid=(N // GB,),
        in_specs=[
            pl.BlockSpec((32, GB, 32), lambda i: (0, i, 0)),
            pl.BlockSpec((160, 896), lambda i: (0, 0)),
            pl.BlockSpec((1, 896), lambda i: (0, 0)),
            pl.BlockSpec((1280, 256), lambda i: (0, 0)),
            pl.BlockSpec((1, 256), lambda i: (0, 0)),
        ],
        out_specs=pl.BlockSpec((20, 5, GB, 256), lambda i: (0, 0, i, 0)),
        compiler_params=pltpu.CompilerParams(
            dimension_semantics=("arbitrary",)),
    )(xb, wbcat, b1t, W2p, b2p)

    fq = f.reshape(100, N, 256)
    w1r = wl1m.reshape(100, 256, 128).astype(jnp.bfloat16)
    Q = 5
    out = pl.pallas_call(
        _mlp_kernel,
        out_shape=jax.ShapeDtypeStruct((N, 10), jnp.float32),
        grid=(100 // Q,),
        in_specs=[
            pl.BlockSpec((Q, N, 256), lambda k: (k, 0, 0)),
            pl.BlockSpec((100, 256, 128), lambda k: (0, 0, 0)),
            pl.BlockSpec((1, 128), lambda k: (0, 0)),
            pl.BlockSpec((128, 10), lambda k: (0, 0)),
            pl.BlockSpec((1, 10), lambda k: (0, 0)),
        ],
        out_specs=pl.BlockSpec((N, 10), lambda k: (0, 0)),
        scratch_shapes=[pltpu.VMEM((N, 128), jnp.float32)],
        compiler_params=pltpu.CompilerParams(
            dimension_semantics=("arbitrary",)),
    )(fq, w1r, bl1r, wl2m, bl2r)
    return out


# speedup vs baseline: 1.0509x; 1.0509x over previous
import math

import jax
import jax.numpy as jnp
from jax.experimental import pallas as pl
from jax.experimental.pallas import tpu as pltpu


def _conv_stage_kernel(x_ref, wb_ref, b1_ref, w2_ref, b2_ref, o_ref):
    GB = x_ref.shape[1]
    x = x_ref[...].astype(jnp.bfloat16)
    x5 = jnp.concatenate([x[kh:kh + 28] for kh in range(5)], axis=-1)
    y1 = jnp.dot(x5.reshape(28 * GB, 160), wb_ref[...],
                 preferred_element_type=jnp.float32)
    y1 = jnp.maximum(y1 + b1_ref[...], 0.0)
    y1 = y1.astype(jnp.bfloat16).reshape(28, GB, 896)

    tg = []
    for g in range(6):
        pg = jnp.concatenate(
            [y1[kh:kh + 24, :, 128 * g:128 * g + 256] for kh in range(5)],
            axis=-1)
        cg = jnp.dot(pg.reshape(24 * GB, 1280), w2_ref[...],
                     preferred_element_type=jnp.float32)
        cg = jnp.maximum(cg + b2_ref[...], 0.0)
        cg = cg.astype(jnp.bfloat16).reshape(24, GB, 256)
        t = cg[0:20]
        for d in range(1, 5):
            t = jnp.maximum(t, cg[d:d + 20])
        tg.append(t)

    for gp in range(5):
        s = jnp.concatenate([tg[gp], tg[gp + 1]], axis=-1)
        chunks = []
        for j in range(4):
            m = s[:, :, 64 * j:64 * j + 64]
            for d in range(1, 5):
                m = jnp.maximum(m, s[:, :, 64 * (j + d):64 * (j + d) + 64])
            chunks.append(m)
        o_ref[:, gp] = jnp.concatenate(chunks, axis=-1)


def _mlp_kernel(f_ref, w1_ref, b1_ref, w2_ref, b2_ref, o_ref, acc_ref):
    k = pl.program_id(0)
    Q = f_ref.shape[0]

    @pl.when(k == 0)
    def _():
        acc_ref[...] = jnp.zeros_like(acc_ref)

    part = jnp.dot(f_ref[0], w1_ref[k * Q],
                   preferred_element_type=jnp.float32)
    for qq in range(1, Q):
        part = part + jnp.dot(f_ref[qq], w1_ref[k * Q + qq],
                              preferred_element_type=jnp.float32)
    acc_ref[...] += part

    @pl.when(k == pl.num_programs(0) - 1)
    def _():
        h = jnp.maximum(acc_ref[...] + b1_ref[...], 0.0)
        logits = jnp.dot(h, w2_ref[...],
                         preferred_element_type=jnp.float32) + b2_ref[...]
        m = jnp.max(logits, axis=1, keepdims=True)
        e = jnp.exp(logits - m)
        o_ref[...] = e / jnp.sum(e, axis=1, keepdims=True)


def kernel(x, Wb, b1t, w2m, b2r, wl1m, bl1r, wl2m, bl2r):
    N = x.shape[0]
    xb = x.reshape(N, 32, 32).transpose(1, 0, 2)
    wbcat = Wb.reshape(160, 896).astype(jnp.bfloat16)

    w25 = w2m.reshape(5, 5, 32, 64)
    segs = [jnp.pad(w25, ((0, 0), (j, 3 - j), (0, 0), (0, 0)))
            for j in range(4)]
    W2p = jnp.stack(segs, axis=3).reshape(1280, 256).astype(jnp.bfloat16)
    b2p = jnp.tile(b2r, (1, 4))

    GB = math.gcd(N, 32)
    f = pl.pallas_call(
        _conv_stage_kernel,
        out_shape=jax.ShapeDtypeStruct((20, 5, N, 256), jnp.bfloat16),
        grid=(N // GB,),
        in_specs=[
            pl.BlockSpec((32, GB, 32), lambda i: (0, i, 0)),
            pl.BlockSpec((160, 896), lambda i: (0, 0)),
            pl.BlockSpec((1, 896), lambda i: (0, 0)),
            pl.BlockSpec((1280, 256), lambda i: (0, 0)),
            pl.BlockSpec((1, 256), lambda i: (0, 0)),
        ],
        out_specs=pl.BlockSpec((20, 5, GB, 256), lambda i: (0, 0, i, 0)),
        compiler_params=pltpu.CompilerParams(
            dimension_semantics=("arbitrary",)),
    )(xb, wbcat, b1t, W2p, b2p)

    fq = f.reshape(100, N, 256)
    w1r = wl1m.reshape(100, 256, 128).astype(jnp.bfloat16)
    Q = 5
    out = pl.pallas_call(
        _mlp_kernel,
        out_shape=jax.ShapeDtypeStruct((N, 10), jnp.float32),
        grid=(100 // Q,),
        in_specs=[
            pl.BlockSpec((Q, N, 256), lambda k: (k, 0, 0)),
            pl.BlockSpec((100, 256, 128), lambda k: (0, 0, 0)),
            pl.BlockSpec((1, 128), lambda k: (0, 0)),
            pl.BlockSpec((128, 10), lambda k: (0, 0)),
            pl.BlockSpec((1, 10), lambda k: (0, 0)),
        ],
        out_specs=pl.BlockSpec((N, 10), lambda k: (0, 0)),
        scratch_shapes=[pltpu.VMEM((N, 128), jnp.float32)],
        compiler_params=pltpu.CompilerParams(
            dimension_semantics=("arbitrary",)),
    )(fq, w1r, bl1r, wl2m, bl2r)
    return out


# speedup vs baseline: 1.0978x; 1.0446x over previous
import math

import jax
import jax.numpy as jnp
from jax.experimental import pallas as pl
from jax.experimental.pallas import tpu as pltpu


def _net_kernel(x_ref, wb_ref, b1_ref, w2_ref, b2_ref, w1_ref, bl1_ref,
                wl2_ref, bl2_ref, o_ref):
    GB = x_ref.shape[1]
    x = x_ref[...].astype(jnp.bfloat16)
    x5 = jnp.concatenate([x[kh:kh + 28] for kh in range(5)], axis=-1)
    y1 = jnp.dot(x5.reshape(28 * GB, 160), wb_ref[...],
                 preferred_element_type=jnp.float32)
    y1 = jnp.maximum(y1 + b1_ref[...], 0.0)
    y1 = y1.astype(jnp.bfloat16).reshape(28, GB, 896)

    tg = []
    for g in range(6):
        pg = jnp.concatenate(
            [y1[kh:kh + 24, :, 128 * g:128 * g + 256] for kh in range(5)],
            axis=-1)
        cg = jnp.dot(pg.reshape(24 * GB, 1280), w2_ref[...],
                     preferred_element_type=jnp.float32)
        cg = jnp.maximum(cg + b2_ref[...], 0.0)
        cg = cg.astype(jnp.bfloat16).reshape(24, GB, 256)
        t = cg[0:20]
        for d in range(1, 5):
            t = jnp.maximum(t, cg[d:d + 20])
        tg.append(t)

    hacc = jnp.zeros((GB, 128), jnp.float32)
    for gp in range(5):
        s = jnp.concatenate([tg[gp], tg[gp + 1]], axis=-1)
        chunks = []
        for j in range(4):
            m = s[:, :, 64 * j:64 * j + 64]
            for d in range(1, 5):
                m = jnp.maximum(m, s[:, :, 64 * (j + d):64 * (j + d) + 64])
            chunks.append(m)
        p2 = jnp.concatenate(chunks, axis=-1)
        for h in range(20):
            hacc = hacc + jnp.dot(p2[h], w1_ref[h * 5 + gp],
                                  preferred_element_type=jnp.float32)

    h1 = jnp.maximum(hacc + bl1_ref[...], 0.0)
    logits = jnp.dot(h1, wl2_ref[...],
                     preferred_element_type=jnp.float32) + bl2_ref[...]
    mx = jnp.max(logits, axis=1, keepdims=True)
    e = jnp.exp(logits - mx)
    o_ref[...] = e / jnp.sum(e, axis=1, keepdims=True)


def kernel(x, Wb, b1t, w2m, b2r, wl1m, bl1r, wl2m, bl2r):
    N = x.shape[0]
    xb = x.reshape(N, 32, 32).transpose(1, 0, 2)
    wbcat = Wb.reshape(160, 896).astype(jnp.bfloat16)

    w25 = w2m.reshape(5, 5, 32, 64)
    segs = [jnp.pad(w25, ((0, 0), (j, 3 - j), (0, 0), (0, 0)))
            for j in range(4)]
    W2p = jnp.stack(segs, axis=3).reshape(1280, 256).astype(jnp.bfloat16)
    b2p = jnp.tile(b2r, (1, 4))
    w1r = wl1m.reshape(100, 256, 128).astype(jnp.bfloat16)

    GB = math.gcd(N, 128)
    out = pl.pallas_call(
        _net_kernel,
        out_shape=jax.ShapeDtypeStruct((N, 10), jnp.float32),
        grid=(N // GB,),
        in_specs=[
            pl.BlockSpec((32, GB, 32), lambda i: (0, i, 0)),
            pl.BlockSpec((160, 896), lambda i: (0, 0)),
            pl.BlockSpec((1, 896), lambda i: (0, 0)),
            pl.BlockSpec((1280, 256), lambda i: (0, 0)),
            pl.BlockSpec((1, 256), lambda i: (0, 0)),
            pl.BlockSpec((100, 256, 128), lambda i: (0, 0, 0)),
            pl.BlockSpec((1, 128), lambda i: (0, 0)),
            pl.BlockSpec((128, 10), lambda i: (0, 0)),
            pl.BlockSpec((1, 10), lambda i: (0, 0)),
        ],
        out_specs=pl.BlockSpec((GB, 10), lambda i: (i, 0)),
        compiler_params=pltpu.CompilerParams(
            dimension_semantics=("arbitrary",)),
    )(xb, wbcat, b1t, W2p, b2p, w1r, bl1r, wl2m, bl2r)
    return out


# speedup vs baseline: 1.1692x; 1.0650x over previous
import math

import jax
import jax.numpy as jnp
from jax.experimental import pallas as pl
from jax.experimental.pallas import tpu as pltpu


def _net_kernel(x_ref, wb_ref, b1_ref, w2_ref, b2_ref, w1_ref, bl1_ref,
                wl2_ref, bl2_ref, o_ref):
    GB = x_ref.shape[1]
    x = x_ref[...].astype(jnp.bfloat16)
    x5 = jnp.concatenate([x[kh:kh + 28] for kh in range(5)], axis=-1)
    y1 = jnp.dot(x5.reshape(28 * GB, 160), wb_ref[...],
                 preferred_element_type=jnp.float32)
    y1 = jnp.maximum(y1 + b1_ref[...], 0.0)
    y1 = y1.astype(jnp.bfloat16).reshape(28, GB, 896)

    tg = []
    for g in range(6):
        pg = jnp.concatenate(
            [y1[kh:kh + 24, :, 128 * g:128 * g + 256] for kh in range(5)],
            axis=-1)
        cg = jnp.dot(pg.reshape(24 * GB, 1280), w2_ref[...],
                     preferred_element_type=jnp.float32)
        cg = jnp.maximum(cg + b2_ref[...], 0.0)
        cg = cg.astype(jnp.bfloat16).reshape(24, GB, 256)
        a = jnp.maximum(cg[0:23], cg[1:24])
        b = jnp.maximum(a[0:21], a[2:23])
        tg.append(jnp.maximum(b[0:20], cg[4:24]))

    hacc = jnp.zeros((GB, 128), jnp.float32)
    for gp in range(5):
        s = jnp.concatenate([tg[gp], tg[gp + 1]], axis=-1)
        a = jnp.maximum(s[:, :, 0:448], s[:, :, 64:512])
        b = jnp.maximum(a[:, :, 0:320], a[:, :, 128:448])
        p2 = jnp.maximum(b[:, :, 0:256], s[:, :, 256:512])
        for h in range(20):
            hacc = hacc + jnp.dot(p2[h], w1_ref[h * 5 + gp],
                                  preferred_element_type=jnp.float32)

    h1 = jnp.maximum(hacc + bl1_ref[...], 0.0)
    logits = jnp.dot(h1, wl2_ref[...],
                     preferred_element_type=jnp.float32) + bl2_ref[...]
    mx = jnp.max(logits, axis=1, keepdims=True)
    e = jnp.exp(logits - mx)
    o_ref[...] = e / jnp.sum(e, axis=1, keepdims=True)


def kernel(x, Wb, b1t, w2m, b2r, wl1m, bl1r, wl2m, bl2r):
    N = x.shape[0]
    xb = x.reshape(N, 32, 32).transpose(1, 0, 2)
    wbcat = Wb.reshape(160, 896).astype(jnp.bfloat16)

    w25 = w2m.reshape(5, 5, 32, 64)
    segs = [jnp.pad(w25, ((0, 0), (j, 3 - j), (0, 0), (0, 0)))
            for j in range(4)]
    W2p = jnp.stack(segs, axis=3).reshape(1280, 256).astype(jnp.bfloat16)
    b2p = jnp.tile(b2r, (1, 4))
    w1r = wl1m.reshape(100, 256, 128).astype(jnp.bfloat16)

    GB = math.gcd(N, 128)
    out = pl.pallas_call(
        _net_kernel,
        out_shape=jax.ShapeDtypeStruct((N, 10), jnp.float32),
        grid=(N // GB,),
        in_specs=[
            pl.BlockSpec((32, GB, 32), lambda i: (0, i, 0)),
            pl.BlockSpec((160, 896), lambda i: (0, 0)),
            pl.BlockSpec((1, 896), lambda i: (0, 0)),
            pl.BlockSpec((1280, 256), lambda i: (0, 0)),
            pl.BlockSpec((1, 256), lambda i: (0, 0)),
            pl.BlockSpec((100, 256, 128), lambda i: (0, 0, 0)),
            pl.BlockSpec((1, 128), lambda i: (0, 0)),
            pl.BlockSpec((128, 10), lambda i: (0, 0)),
            pl.BlockSpec((1, 10), lambda i: (0, 0)),
        ],
        out_specs=pl.BlockSpec((GB, 10), lambda i: (i, 0)),
        compiler_params=pltpu.CompilerParams(
            dimension_semantics=("arbitrary",)),
    )(xb, wbcat, b1t, W2p, b2p, w1r, bl1r, wl2m, bl2r)
    return out
